# baseline (speedup 1.0000x reference)
import functools

import jax
import jax.numpy as jnp
from jax import lax
from jax.experimental import pallas as pl
from jax.experimental.pallas import tpu as pltpu
from jax.experimental.pallas import tpu_sc as plsc

N = 10000
E = 320000
D = 128
G = 64

NC = 2
NS = 16
NW = NC * NS
CL = 128
CW = 80
IB = 16
NB = CW // IB
EPAD = NW * CW * CL
RPT = 632
NPAD = NS * RPT
RB = 2000

CB = 8

_mesh = plsc.VectorSubcoreMesh(core_axis_name="c", subcore_axis_name="s")


def _seg_sum_sc(xp, src3, dst3, with_counts=False):
    out_type = [jax.ShapeDtypeStruct((NC, NPAD, D), jnp.float32)]
    scratch = [
        pltpu.VMEM((2, IB, CL), jnp.int32),
        pltpu.VMEM((2, IB, CL), jnp.int32),
        pltpu.VMEM((CL, D), jnp.float32),
        pltpu.VMEM((CL, D), jnp.float32),
        pltpu.VMEM_SHARED((NPAD, D), jnp.float32),
        pltpu.SemaphoreType.DMA,
        pltpu.SemaphoreType.DMA,
        pltpu.SemaphoreType.DMA,
        pltpu.SemaphoreType.DMA,
    ]
    if with_counts:
        out_type.append(jax.ShapeDtypeStruct((NC, NPAD, CB), jnp.float32))
        scratch += [
            pltpu.VMEM((CL, CB), jnp.float32),
            pltpu.VMEM_SHARED((NPAD, CB), jnp.float32),
            pltpu.SemaphoreType.DMA,
        ]

    def body(x_hbm, src_hbm, dst_hbm, *rest):
        if with_counts:
            (ones_hbm, zeros_hbm, out_hbm, cnt_hbm, src_r, dst_r, rows_a,
             rows_b, acc_s, sem_a, sem_b, sem_is, sem_id,
             ones_v, cacc_s, sem_c) = rest
        else:
            (out_hbm, src_r, dst_r, rows_a, rows_b,
             acc_s, sem_a, sem_b, sem_is, sem_id) = rest
        c = lax.axis_index("c")
        s = lax.axis_index("s")
        wid = s * NC + c

        pltpu.sync_copy(src_hbm.at[wid, pl.ds(0, IB)], src_r.at[0])
        pltpu.sync_copy(dst_hbm.at[wid, pl.ds(0, IB)], dst_r.at[0])

        @pl.loop(0, CL)
        def _(r):
            @pl.loop(0, D // 16)
            def _(k):
                rows_a[r, pl.ds(k * 16, 16)] = jnp.zeros((16,), jnp.float32)

        base = s * RPT
        for kk in range(RPT // CL):
            pltpu.sync_copy(rows_a, acc_s.at[pl.ds(base + kk * CL, CL)])
        rem = RPT % CL
        if rem:
            pltpu.sync_copy(rows_a.at[pl.ds(0, rem)],
                            acc_s.at[pl.ds(base + RPT - rem, rem)])
        if with_counts:
            pltpu.sync_copy(ones_hbm, ones_v)
            pltpu.sync_copy(zeros_hbm.at[pl.ds(base, RPT)],
                            cacc_s.at[pl.ds(base, RPT)])
        plsc.subcore_barrier()

        @pl.loop(0, NB)
        def _(b):
            sb = b % 2

            @pl.when(b + 1 < NB)
            def _():
                pltpu.async_copy(src_hbm.at[wid, pl.ds((b + 1) * IB, IB)],
                                 src_r.at[1 - sb], sem_is)
                pltpu.async_copy(dst_hbm.at[wid, pl.ds((b + 1) * IB, IB)],
                                 dst_r.at[1 - sb], sem_id)

            pltpu.async_copy(x_hbm.at[src_r.at[sb, 0]], rows_a, sem_a)

            @pl.loop(0, IB // 2)
            def _(m):
                j = m * 2
                db = pltpu.async_copy(x_hbm.at[src_r.at[sb, j + 1]],
                                     rows_b, sem_b)
                pltpu.make_async_copy(x_hbm.at[src_r.at[sb, j]],
                                      rows_a, sem_a).wait()
                pltpu.sync_copy(rows_a, acc_s.at[dst_r.at[sb, j]], add=True)

                @pl.when(j + 2 < IB)
                def _():
                    pltpu.async_copy(x_hbm.at[src_r.at[sb, j + 2]],
                                     rows_a, sem_a)

                if with_counts:
                    pltpu.async_copy(ones_v, cacc_s.at[dst_r.at[sb, j]],
                                     sem_c, add=True)
                db.wait()
                pltpu.sync_copy(rows_b, acc_s.at[dst_r.at[sb, j + 1]], add=True)
                if with_counts:
                    pltpu.async_copy(ones_v, cacc_s.at[dst_r.at[sb, j + 1]],
                                     sem_c, add=True)

            if with_counts:
                @pl.loop(0, IB)
                def _(_):
                    pltpu.make_async_copy(
                        ones_v, cacc_s.at[dst_r.at[sb, 0]], sem_c).wait()

            @pl.when(b + 1 < NB)
            def _():
                pltpu.make_async_copy(src_hbm.at[wid, pl.ds(0, IB)],
                                      src_r.at[0], sem_is).wait()
                pltpu.make_async_copy(dst_hbm.at[wid, pl.ds(0, IB)],
                                      dst_r.at[0], sem_id).wait()

        plsc.subcore_barrier()
        pltpu.sync_copy(acc_s.at[pl.ds(base, RPT)],
                        out_hbm.at[c, pl.ds(base, RPT)])
        if with_counts:
            pltpu.sync_copy(cacc_s.at[pl.ds(base, RPT)],
                            cnt_hbm.at[c, pl.ds(base, RPT)])

    cp = (pltpu.CompilerParams(use_tc_tiling_on_sc=False)
          if with_counts else None)
    k = pl.kernel(body,
                  out_type=tuple(out_type) if with_counts else out_type[0],
                  mesh=_mesh, scratch_types=scratch, compiler_params=cp)
    if with_counts:
        return k(xp, src3, dst3, jnp.ones((CL, CB), jnp.float32),
                 jnp.zeros((NPAD, CB), jnp.float32))
    return k(xp, src3, dst3)


_DOT = functools.partial(lax.dot_general,
                         dimension_numbers=(((1,), (0,)), ((), ())),
                         preferred_element_type=jnp.float32,
                         precision=lax.Precision.HIGHEST)


def _linear_body(x_ref, w_ref, b1_ref, b2_ref, o_ref):
    o_ref[...] = _DOT(x_ref[...], w_ref[...]) + b1_ref[...] + b2_ref[...]


def _tc_linear(xp, W, b1, b2):
    grid = (N // RB,)
    return pl.pallas_call(
        _linear_body,
        grid=grid,
        in_specs=[
            pl.BlockSpec((RB, D), lambda i: (i, 0)),
            pl.BlockSpec((D, D), lambda i: (0, 0)),
            pl.BlockSpec((1, D), lambda i: (0, 0)),
            pl.BlockSpec((1, D), lambda i: (0, 0)),
        ],
        out_specs=pl.BlockSpec((RB, D), lambda i: (i, 0)),
        out_shape=jax.ShapeDtypeStruct((N, D), jnp.float32),
    )(xp, W, b1.reshape(1, D), b2.reshape(1, D))


def _combine_linear_body(sa_ref, sb_ref, ca_ref, cb_ref, xr_ref, w_ref,
                         wr_ref, b1_ref, b2_ref, h_ref, hr_ref):
    cnt = jnp.maximum(ca_ref[...] + cb_ref[...], 1.0)
    neigh = (sa_ref[...] + sb_ref[...]) / cnt
    h = jnp.maximum(_DOT(neigh, w_ref[...]) + xr_ref[...], 0.0)
    h_ref[...] = h
    hr_ref[...] = _DOT(h, wr_ref[...]) + b1_ref[...] + b2_ref[...]


def _tc_combine_linear(Sa, Sb, ca, cb, xr, W_l, W_r, b1, b2):
    grid = (N // RB,)
    return pl.pallas_call(
        _combine_linear_body,
        grid=grid,
        in_specs=[
            pl.BlockSpec((RB, D), lambda i: (i, 0)),
            pl.BlockSpec((RB, D), lambda i: (i, 0)),
            pl.BlockSpec((RB, 1), lambda i: (i, 0)),
            pl.BlockSpec((RB, 1), lambda i: (i, 0)),
            pl.BlockSpec((RB, D), lambda i: (i, 0)),
            pl.BlockSpec((D, D), lambda i: (0, 0)),
            pl.BlockSpec((D, D), lambda i: (0, 0)),
            pl.BlockSpec((1, D), lambda i: (0, 0)),
            pl.BlockSpec((1, D), lambda i: (0, 0)),
        ],
        out_specs=[pl.BlockSpec((RB, D), lambda i: (i, 0)),
                   pl.BlockSpec((RB, D), lambda i: (i, 0))],
        out_shape=[jax.ShapeDtypeStruct((N, D), jnp.float32),
                   jax.ShapeDtypeStruct((N, D), jnp.float32)],
    )(Sa, Sb, ca, cb, xr, W_l, W_r, b1.reshape(1, D), b2.reshape(1, D))


def _combine_readout_body(sa_ref, sb_ref, ca_ref, cb_ref, xr_ref, w_ref,
                          b_ref, o_ref, acc_ref, cnt_ref):
    i = pl.program_id(0)

    @pl.when(i == 0)
    def _():
        acc_ref[...] = jnp.zeros_like(acc_ref)
        cnt_ref[...] = jnp.zeros_like(cnt_ref)

    cnt = jnp.maximum(ca_ref[...] + cb_ref[...], 1.0)
    neigh = (sa_ref[...] + sb_ref[...]) / cnt
    h2 = jnp.maximum(_DOT(neigh, w_ref[...]) + xr_ref[...], 0.0)

    oh = (b_ref[...] == lax.broadcasted_iota(jnp.int32, (RB, G), 1))
    ohf = oh.astype(jnp.float32)
    acc_ref[...] += lax.dot_general(
        ohf, h2, dimension_numbers=(((0,), (0,)), ((), ())),
        preferred_element_type=jnp.float32, precision=lax.Precision.HIGHEST)
    cnt_ref[...] += jnp.broadcast_to(
        jnp.sum(ohf, axis=0).reshape(G, 1), (G, D))

    @pl.when(i == N // RB - 1)
    def _():
        o_ref[...] = acc_ref[...] / jnp.maximum(cnt_ref[...], 1.0)


def _tc_combine_readout(Sa, Sb, ca, cb, xr, W_l, batch_2d):
    grid = (N // RB,)
    return pl.pallas_call(
        _combine_readout_body,
        grid=grid,
        in_specs=[
            pl.BlockSpec((RB, D), lambda i: (i, 0)),
            pl.BlockSpec((RB, D), lambda i: (i, 0)),
            pl.BlockSpec((RB, 1), lambda i: (i, 0)),
            pl.BlockSpec((RB, 1), lambda i: (i, 0)),
            pl.BlockSpec((RB, D), lambda i: (i, 0)),
            pl.BlockSpec((D, D), lambda i: (0, 0)),
            pl.BlockSpec((RB, 1), lambda i: (i, 0)),
        ],
        out_specs=pl.BlockSpec((G, D), lambda i: (0, 0)),
        out_shape=jax.ShapeDtypeStruct((G, D), jnp.float32),
        scratch_shapes=[pltpu.VMEM((G, D), jnp.float32),
                        pltpu.VMEM((G, D), jnp.float32)],
    )(Sa, Sb, ca, cb, xr, W_l, batch_2d)


def kernel(x, edge_index, batch, W_l0, b_l0, W_r0, b_r0, W_l1, b_l1, W_r1, b_r1):
    pad_e = EPAD - E
    ar = jnp.arange(pad_e, dtype=jnp.int32)
    src_pad = jnp.concatenate([edge_index[0], ar % N])
    dst_pad = jnp.concatenate([edge_index[1], N + ar % (NPAD - N)])
    src3 = src_pad.reshape(NW, CW, CL)
    dst3 = dst_pad.reshape(NW, CW, CL)
    batch_2d = batch.reshape(N, 1)

    S0, cnt = _seg_sum_sc(x, src3, dst3, with_counts=True)
    ca = cnt[0, :, 0:1]
    cb = cnt[1, :, 0:1]
    xr0 = _tc_linear(x, W_r0, b_l0, b_r0)
    h, hr1 = _tc_combine_linear(S0[0], S0[1], ca, cb, xr0, W_l0,
                                W_r1, b_l1, b_r1)

    S1 = _seg_sum_sc(h, src3, dst3)
    return _tc_combine_readout(S1[0], S1[1], ca, cb, hr1, W_l1, batch_2d)


# speedup vs baseline: 1.0192x; 1.0192x over previous
import functools

import jax
import jax.numpy as jnp
from jax import lax
from jax.experimental import pallas as pl
from jax.experimental.pallas import tpu as pltpu
from jax.experimental.pallas import tpu_sc as plsc

N = 10000
E = 320000
D = 128
G = 64

NC = 2
NS = 16
NW = NC * NS
CL = 128
CW = 80
IB = 16
NB = CW // IB
EPAD = NW * CW * CL
RPT = 632
NPAD = NS * RPT
RB = 2000

CB = 8

_mesh = plsc.VectorSubcoreMesh(core_axis_name="c", subcore_axis_name="s")


def _seg_sum_sc(xp, src3, dst3, with_counts=False):
    out_type = [jax.ShapeDtypeStruct((NC, NPAD, D), jnp.float32)]
    scratch = [
        pltpu.VMEM((2, IB, CL), jnp.int32),
        pltpu.VMEM((2, IB, CL), jnp.int32),
        pltpu.VMEM((CL, D), jnp.float32),
        pltpu.VMEM((CL, D), jnp.float32),
        pltpu.VMEM_SHARED((NPAD, D), jnp.float32),
        pltpu.SemaphoreType.DMA,
        pltpu.SemaphoreType.DMA,
        pltpu.SemaphoreType.DMA,
        pltpu.SemaphoreType.DMA,
    ]
    if with_counts:
        out_type.append(jax.ShapeDtypeStruct((NC, NPAD, CB), jnp.float32))
        scratch += [
            pltpu.VMEM((CL, CB), jnp.float32),
            pltpu.VMEM_SHARED((NPAD, CB), jnp.float32),
            pltpu.SemaphoreType.DMA,
        ]

    def body(x_hbm, src_hbm, dst_hbm, *rest):
        if with_counts:
            (ones_hbm, zeros_hbm, out_hbm, cnt_hbm, src_r, dst_r, rows_a,
             rows_b, acc_s, sem_a, sem_b, sem_is, sem_id,
             ones_v, cacc_s, sem_c) = rest
        else:
            (out_hbm, src_r, dst_r, rows_a, rows_b,
             acc_s, sem_a, sem_b, sem_is, sem_id) = rest
        c = lax.axis_index("c")
        s = lax.axis_index("s")
        wid = s * NC + c

        pltpu.sync_copy(src_hbm.at[wid, pl.ds(0, IB)], src_r.at[0])
        pltpu.sync_copy(dst_hbm.at[wid, pl.ds(0, IB)], dst_r.at[0])

        @pl.loop(0, CL)
        def _(r):
            @pl.loop(0, D // 16)
            def _(k):
                rows_a[r, pl.ds(k * 16, 16)] = jnp.zeros((16,), jnp.float32)

        base = s * RPT
        for kk in range(RPT // CL):
            pltpu.sync_copy(rows_a, acc_s.at[pl.ds(base + kk * CL, CL)])
        rem = RPT % CL
        if rem:
            pltpu.sync_copy(rows_a.at[pl.ds(0, rem)],
                            acc_s.at[pl.ds(base + RPT - rem, rem)])
        if with_counts:
            pltpu.sync_copy(ones_hbm, ones_v)
            pltpu.sync_copy(zeros_hbm.at[pl.ds(base, RPT)],
                            cacc_s.at[pl.ds(base, RPT)])
        plsc.subcore_barrier()

        @pl.loop(0, NB)
        def _(b):
            sb = b % 2

            @pl.when(b + 1 < NB)
            def _():
                pltpu.async_copy(src_hbm.at[wid, pl.ds((b + 1) * IB, IB)],
                                 src_r.at[1 - sb], sem_is)
                pltpu.async_copy(dst_hbm.at[wid, pl.ds((b + 1) * IB, IB)],
                                 dst_r.at[1 - sb], sem_id)

            pltpu.async_copy(x_hbm.at[src_r.at[sb, 0]], rows_a, sem_a)

            @pl.loop(0, IB // 2)
            def _(m):
                j = m * 2
                db = pltpu.async_copy(x_hbm.at[src_r.at[sb, j + 1]],
                                     rows_b, sem_b)
                pltpu.make_async_copy(x_hbm.at[src_r.at[sb, j]],
                                      rows_a, sem_a).wait()
                pltpu.sync_copy(rows_a, acc_s.at[dst_r.at[sb, j]], add=True)

                @pl.when(j + 2 < IB)
                def _():
                    pltpu.async_copy(x_hbm.at[src_r.at[sb, j + 2]],
                                     rows_a, sem_a)

                if with_counts:
                    pltpu.async_copy(ones_v, cacc_s.at[dst_r.at[sb, j]],
                                     sem_c, add=True)
                db.wait()
                pltpu.sync_copy(rows_b, acc_s.at[dst_r.at[sb, j + 1]], add=True)
                if with_counts:
                    pltpu.async_copy(ones_v, cacc_s.at[dst_r.at[sb, j + 1]],
                                     sem_c, add=True)

            if with_counts:
                @pl.loop(0, IB)
                def _(_):
                    pltpu.make_async_copy(
                        ones_v, cacc_s.at[dst_r.at[sb, 0]], sem_c).wait()

            @pl.when(b + 1 < NB)
            def _():
                pltpu.make_async_copy(src_hbm.at[wid, pl.ds(0, IB)],
                                      src_r.at[0], sem_is).wait()
                pltpu.make_async_copy(dst_hbm.at[wid, pl.ds(0, IB)],
                                      dst_r.at[0], sem_id).wait()

        plsc.subcore_barrier()
        pltpu.sync_copy(acc_s.at[pl.ds(base, RPT)],
                        out_hbm.at[c, pl.ds(base, RPT)])
        if with_counts:
            pltpu.sync_copy(cacc_s.at[pl.ds(base, RPT)],
                            cnt_hbm.at[c, pl.ds(base, RPT)])

    cp = (pltpu.CompilerParams(use_tc_tiling_on_sc=False)
          if with_counts else None)
    k = pl.kernel(body,
                  out_type=tuple(out_type) if with_counts else out_type[0],
                  mesh=_mesh, scratch_types=scratch, compiler_params=cp)
    if with_counts:
        return k(xp, src3, dst3, jnp.ones((CL, CB), jnp.float32),
                 jnp.zeros((NPAD, CB), jnp.float32))
    return k(xp, src3, dst3)


_DOT = functools.partial(lax.dot_general,
                         dimension_numbers=(((1,), (0,)), ((), ())),
                         preferred_element_type=jnp.float32,
                         precision=lax.Precision.HIGHEST)


def _linear_body(x_ref, w_ref, b1_ref, b2_ref, o_ref):
    o_ref[...] = _DOT(x_ref[...], w_ref[...]) + b1_ref[...] + b2_ref[...]


def _tc_linear(xp, W, b1, b2):
    grid = (N // RB,)
    return pl.pallas_call(
        _linear_body,
        grid=grid,
        in_specs=[
            pl.BlockSpec((RB, D), lambda i: (i, 0)),
            pl.BlockSpec((D, D), lambda i: (0, 0)),
            pl.BlockSpec((1, D), lambda i: (0, 0)),
            pl.BlockSpec((1, D), lambda i: (0, 0)),
        ],
        out_specs=pl.BlockSpec((RB, D), lambda i: (i, 0)),
        out_shape=jax.ShapeDtypeStruct((N, D), jnp.float32),
    )(xp, W, b1.reshape(1, D), b2.reshape(1, D))


def _combine_body(sa_ref, sb_ref, ca_ref, cb_ref, xr_ref, w_ref, o_ref):
    cnt = jnp.maximum(ca_ref[...] + cb_ref[...], 1.0)
    neigh = (sa_ref[...] + sb_ref[...]) / cnt
    o_ref[...] = jnp.maximum(_DOT(neigh, w_ref[...]) + xr_ref[...], 0.0)


def _tc_combine(Sa, Sb, ca, cb, xr, W_l):
    grid = (N // RB,)
    return pl.pallas_call(
        _combine_body,
        grid=grid,
        in_specs=[
            pl.BlockSpec((RB, D), lambda i: (i, 0)),
            pl.BlockSpec((RB, D), lambda i: (i, 0)),
            pl.BlockSpec((RB, 1), lambda i: (i, 0)),
            pl.BlockSpec((RB, 1), lambda i: (i, 0)),
            pl.BlockSpec((RB, D), lambda i: (i, 0)),
            pl.BlockSpec((D, D), lambda i: (0, 0)),
        ],
        out_specs=pl.BlockSpec((RB, D), lambda i: (i, 0)),
        out_shape=jax.ShapeDtypeStruct((N, D), jnp.float32),
    )(Sa, Sb, ca, cb, xr, W_l)


def _combine_readout_body(sa_ref, sb_ref, ca_ref, cb_ref, xr_ref, w_ref,
                          b_ref, o_ref, acc_ref, cnt_ref):
    i = pl.program_id(0)

    @pl.when(i == 0)
    def _():
        acc_ref[...] = jnp.zeros_like(acc_ref)
        cnt_ref[...] = jnp.zeros_like(cnt_ref)

    cnt = jnp.maximum(ca_ref[...] + cb_ref[...], 1.0)
    neigh = (sa_ref[...] + sb_ref[...]) / cnt
    h2 = jnp.maximum(_DOT(neigh, w_ref[...]) + xr_ref[...], 0.0)

    oh = (b_ref[...] == lax.broadcasted_iota(jnp.int32, (RB, G), 1))
    ohf = oh.astype(jnp.float32)
    acc_ref[...] += lax.dot_general(
        ohf, h2, dimension_numbers=(((0,), (0,)), ((), ())),
        preferred_element_type=jnp.float32, precision=lax.Precision.HIGHEST)
    cnt_ref[...] += jnp.broadcast_to(
        jnp.sum(ohf, axis=0).reshape(G, 1), (G, D))

    @pl.when(i == N // RB - 1)
    def _():
        o_ref[...] = acc_ref[...] / jnp.maximum(cnt_ref[...], 1.0)


def _tc_combine_readout(Sa, Sb, ca, cb, xr, W_l, batch_2d):
    grid = (N // RB,)
    return pl.pallas_call(
        _combine_readout_body,
        grid=grid,
        in_specs=[
            pl.BlockSpec((RB, D), lambda i: (i, 0)),
            pl.BlockSpec((RB, D), lambda i: (i, 0)),
            pl.BlockSpec((RB, 1), lambda i: (i, 0)),
            pl.BlockSpec((RB, 1), lambda i: (i, 0)),
            pl.BlockSpec((RB, D), lambda i: (i, 0)),
            pl.BlockSpec((D, D), lambda i: (0, 0)),
            pl.BlockSpec((RB, 1), lambda i: (i, 0)),
        ],
        out_specs=pl.BlockSpec((G, D), lambda i: (0, 0)),
        out_shape=jax.ShapeDtypeStruct((G, D), jnp.float32),
        scratch_shapes=[pltpu.VMEM((G, D), jnp.float32),
                        pltpu.VMEM((G, D), jnp.float32)],
    )(Sa, Sb, ca, cb, xr, W_l, batch_2d)


def kernel(x, edge_index, batch, W_l0, b_l0, W_r0, b_r0, W_l1, b_l1, W_r1, b_r1):
    pad_e = EPAD - E
    ar = jnp.arange(pad_e, dtype=jnp.int32)
    src_pad = jnp.concatenate([edge_index[0], ar % N])
    dst_pad = jnp.concatenate([edge_index[1], N + ar % (NPAD - N)])
    src3 = src_pad.reshape(NW, CW, CL)
    dst3 = dst_pad.reshape(NW, CW, CL)
    batch_2d = batch.reshape(N, 1)

    S0, cnt = _seg_sum_sc(x, src3, dst3, with_counts=True)
    ca = cnt[0, :, 0:1]
    cb = cnt[1, :, 0:1]
    xr0 = _tc_linear(x, W_r0, b_l0, b_r0)
    h = _tc_combine(S0[0], S0[1], ca, cb, xr0, W_l0)

    S1 = _seg_sum_sc(h, src3, dst3)
    hr1 = _tc_linear(h, W_r1, b_l1, b_r1)
    return _tc_combine_readout(S1[0], S1[1], ca, cb, hr1, W_l1, batch_2d)
